# baseline (speedup 1.0000x reference)
import functools

import jax
import jax.numpy as jnp
from jax import lax
from jax.experimental import pallas as pl
from jax.experimental.pallas import tpu as pltpu
from jax.experimental.pallas import tpu_sc as plsc

_N = 10000
_E = 160000
_B = 125
_PB = 40
_NPH = 2
_CB = 48
_NP = 10112
_RPT = 632
_BM = 1000


def _mm_body(x_ref, w_ref, o_ref):
    o_ref[...] = jnp.dot(x_ref[...], w_ref[...],
                         preferred_element_type=jnp.float32)


def _mm(x, w):
    m, k = x.shape
    n = w.shape[1]
    return pl.pallas_call(
        _mm_body,
        grid=(m // _BM,),
        in_specs=[pl.BlockSpec((_BM, k), lambda i: (i, 0)),
                  pl.BlockSpec((k, n), lambda i: (0, 0))],
        out_specs=pl.BlockSpec((_BM, n), lambda i: (i, 0)),
        out_shape=jax.ShapeDtypeStruct((m, n), jnp.float32),
    )(x, w)


def _mm_bias_relu_body(x_ref, b_ref, w_ref, o_ref):
    h = jnp.maximum(x_ref[...] + b_ref[...], 0.0)
    o_ref[...] = jnp.dot(h, w_ref[...], preferred_element_type=jnp.float32)


def _mm_bias_relu(x, b, w):
    m, k = x.shape
    n = w.shape[1]
    return pl.pallas_call(
        _mm_bias_relu_body,
        grid=(m // _BM,),
        in_specs=[pl.BlockSpec((_BM, k), lambda i: (i, 0)),
                  pl.BlockSpec((1, k), lambda i: (0, 0)),
                  pl.BlockSpec((k, n), lambda i: (0, 0))],
        out_specs=pl.BlockSpec((_BM, n), lambda i: (i, 0)),
        out_shape=jax.ShapeDtypeStruct((m, n), jnp.float32),
    )(x, b.reshape(1, k), w)


def _logsoftmax_bias_body(x_ref, b_ref, o_ref):
    h = x_ref[...] + b_ref[...]
    m = jnp.max(h, axis=1, keepdims=True)
    e = jnp.exp(h - m)
    s = jnp.sum(e, axis=1, keepdims=True)
    o_ref[...] = h - m - jnp.log(s)


def _logsoftmax_bias(x, b):
    m, n = x.shape
    return pl.pallas_call(
        _logsoftmax_bias_body,
        grid=(m // _BM,),
        in_specs=[pl.BlockSpec((_BM, n), lambda i: (i, 0)),
                  pl.BlockSpec((1, n), lambda i: (0, 0))],
        out_specs=pl.BlockSpec((_BM, n), lambda i: (i, 0)),
        out_shape=jax.ShapeDtypeStruct((m, n), jnp.float32),
    )(x, b.reshape(1, n))


_sc_mesh = plsc.VectorSubcoreMesh(core_axis_name="c", subcore_axis_name="s",
                                  num_cores=2)


_sc_scratch = [
    pltpu.VMEM_SHARED((_NP, 128), jnp.float32),
    pltpu.VMEM((_CB, _B), jnp.int32),
    pltpu.VMEM((_PB, _B), jnp.int32),
    pltpu.VMEM((_B, 128), jnp.float32),
    pltpu.VMEM((_B, 128), jnp.float32),
    pltpu.SemaphoreType.DMA,
    pltpu.SemaphoreType.DMA,
]


def _sc_spmm_body(sup2, col2, row3, zrows, out2,
                  acc, colbuf, rowbuf, ga, gb, sema, semb):
    cid = lax.axis_index("c")
    sid = lax.axis_index("s")
    base = sid * _RPT
    pltpu.sync_copy(zrows, acc.at[pl.ds(base, _RPT)])
    plsc.subcore_barrier()

    for p in range(_NPH):
        pltpu.sync_copy(col2.at[cid, sid, p], colbuf)
        pltpu.sync_copy(row3.at[sid, p], rowbuf)
        pltpu.async_copy(sup2.at[colbuf.at[0]], ga, sema)

        def body(j2, carry):
            j = 2 * j2
            pltpu.async_copy(sup2.at[colbuf.at[j + 1]], gb, semb)
            pltpu.make_async_copy(sup2.at[colbuf.at[j]], ga, sema).wait()
            pltpu.sync_copy(ga, acc.at[rowbuf.at[j]], add=True)
            pltpu.async_copy(sup2.at[colbuf.at[j + 2]], ga, sema)
            pltpu.make_async_copy(sup2.at[colbuf.at[j + 1]], gb, semb).wait()
            pltpu.sync_copy(gb, acc.at[rowbuf.at[j + 1]], add=True)
            return carry

        lax.fori_loop(0, _PB // 2, body, 0)
        pltpu.make_async_copy(sup2.at[colbuf.at[_PB]], ga, sema).wait()

    plsc.subcore_barrier()
    pltpu.sync_copy(acc.at[pl.ds(base, _RPT)],
                    out2.at[cid, pl.ds(base, _RPT)])


_sc_spmm = pl.kernel(
    _sc_spmm_body,
    out_type=jax.ShapeDtypeStruct((2, _NP, 128), jnp.float32),
    mesh=_sc_mesh,
    scratch_types=_sc_scratch,
)


def _spmm(sup, col2, row3, zrows):
    d = sup.shape[1]
    parts = []
    for c0 in range(0, d, 256):
        sup2 = jnp.concatenate([sup[:, c0:c0 + 128],
                                sup[:, c0 + 128:c0 + 256]], axis=0)
        out2 = _sc_spmm(sup2, col2, row3, zrows)
        parts.append(jnp.concatenate([out2[0, :_N], out2[1, :_N]], axis=1))
    return parts[0] if len(parts) == 1 else jnp.concatenate(parts, axis=1)


def kernel(edges, features, W1, b1, W2, b2, W3, b3):
    row = edges[0].astype(jnp.int32)
    col = edges[1].astype(jnp.int32)
    col2 = jnp.stack([col, col + _N]).reshape(2, 16, _NPH, _PB, _B)
    col2 = jnp.concatenate(
        [col2, jnp.zeros((2, 16, _NPH, _CB - _PB, _B), jnp.int32)], axis=3)
    row3 = row.reshape(16, _NPH, _PB, _B)
    zrows = jnp.zeros((_RPT, 128), jnp.float32)

    sup = _mm(features, W1)
    agg = _spmm(sup, col2, row3, zrows)
    sup = _mm_bias_relu(agg, b1, W2)
    agg = _spmm(sup, col2, row3, zrows)
    sup = _mm_bias_relu(agg, b2, W3)
    agg = _spmm(sup, col2, row3, zrows)
    return _logsoftmax_bias(agg, b3)


# speedup vs baseline: 2.0031x; 2.0031x over previous
import jax
import jax.numpy as jnp
from jax import lax
from jax.experimental import pallas as pl
from jax.experimental.pallas import tpu as pltpu
from jax.experimental.pallas import tpu_sc as plsc

_N = 10000
_E = 160000
_B = 125
_NB = 80
_NP = 10112
_RPT = 632
_BM = 1000


def _mm_body(x_ref, w_ref, o_ref):
    o_ref[...] = jnp.dot(x_ref[...], w_ref[...],
                         preferred_element_type=jnp.float32)


def _mm(x, w):
    m, k = x.shape
    n = w.shape[1]
    return pl.pallas_call(
        _mm_body,
        grid=(m // _BM,),
        in_specs=[pl.BlockSpec((_BM, k), lambda i: (i, 0)),
                  pl.BlockSpec((k, n), lambda i: (0, 0))],
        out_specs=pl.BlockSpec((_BM, n), lambda i: (i, 0)),
        out_shape=jax.ShapeDtypeStruct((m, n), jnp.float32),
    )(x, w)


def _mm_bias_relu_body(x_ref, b_ref, w_ref, o_ref):
    h = jnp.maximum(x_ref[...] + b_ref[...], 0.0)
    o_ref[...] = jnp.dot(h, w_ref[...], preferred_element_type=jnp.float32)


def _mm_bias_relu(x, b, w):
    m, k = x.shape
    n = w.shape[1]
    return pl.pallas_call(
        _mm_bias_relu_body,
        grid=(m // _BM,),
        in_specs=[pl.BlockSpec((_BM, k), lambda i: (i, 0)),
                  pl.BlockSpec((1, k), lambda i: (0, 0)),
                  pl.BlockSpec((k, n), lambda i: (0, 0))],
        out_specs=pl.BlockSpec((_BM, n), lambda i: (i, 0)),
        out_shape=jax.ShapeDtypeStruct((m, n), jnp.float32),
    )(x, b.reshape(1, k), w)


def _logsoftmax_bias_body(x_ref, b_ref, o_ref):
    h = x_ref[...] + b_ref[...]
    m = jnp.max(h, axis=1, keepdims=True)
    e = jnp.exp(h - m)
    s = jnp.sum(e, axis=1, keepdims=True)
    o_ref[...] = h - m - jnp.log(s)


def _logsoftmax_bias(x, b):
    m, n = x.shape
    return pl.pallas_call(
        _logsoftmax_bias_body,
        grid=(m // _BM,),
        in_specs=[pl.BlockSpec((_BM, n), lambda i: (i, 0)),
                  pl.BlockSpec((1, n), lambda i: (0, 0))],
        out_specs=pl.BlockSpec((_BM, n), lambda i: (i, 0)),
        out_shape=jax.ShapeDtypeStruct((m, n), jnp.float32),
    )(x, b.reshape(1, n))


_sc_mesh = plsc.VectorSubcoreMesh(core_axis_name="c", subcore_axis_name="s",
                                  num_cores=2)

_sc_scratch = [
    pltpu.VMEM_SHARED((_NP, 128), jnp.float32),
    pltpu.VMEM((_NB, _B), jnp.int32),
    pltpu.VMEM((_NB, _B), jnp.int32),
    pltpu.VMEM((_B, 128), jnp.float32),
    pltpu.SemaphoreType.DMA,
]


def _make_sc_spmm(npair):
    def body(sup4, col4, row3, zrows, out4,
             acc, colbuf, rowbuf, gbuf, sem):
        cid = lax.axis_index("c")
        sid = lax.axis_index("s")
        base = sid * _RPT
        for pair in range(npair):
            pltpu.sync_copy(zrows, acc.at[pl.ds(base, _RPT)])
            pltpu.sync_copy(col4.at[cid, pair, sid], colbuf)
            pltpu.sync_copy(row3.at[sid], rowbuf)
            plsc.subcore_barrier()

            def lbody(j, carry):
                pltpu.async_copy(sup4.at[colbuf.at[j]], gbuf, sem).wait()
                pltpu.sync_copy(gbuf, acc.at[rowbuf.at[j]], add=True)
                return carry

            lax.fori_loop(0, _NB, lbody, 0)
            plsc.subcore_barrier()
            pltpu.sync_copy(acc.at[pl.ds(base, _RPT)],
                            out4.at[2 * pair + cid, pl.ds(base, _RPT)])
            if pair + 1 < npair:
                plsc.subcore_barrier()

    return pl.kernel(
        body,
        out_type=jax.ShapeDtypeStruct((2 * npair, _NP, 128), jnp.float32),
        mesh=_sc_mesh,
        scratch_types=_sc_scratch,
    )


_sc_spmm_2 = _make_sc_spmm(2)
_sc_spmm_1 = _make_sc_spmm(1)


def _spmm(sup, col4, row3, zrows):
    d = sup.shape[1]
    nchunk = d // 128
    sup4 = jnp.concatenate(
        [sup[:, k * 128:(k + 1) * 128] for k in range(nchunk)], axis=0)
    fn = _sc_spmm_2 if nchunk == 4 else _sc_spmm_1
    out4 = fn(sup4, col4[:, :nchunk // 2], row3, zrows)
    return jnp.concatenate([out4[k, :_N] for k in range(nchunk)], axis=1)


def kernel(edges, features, W1, b1, W2, b2, W3, b3):
    row = edges[0].astype(jnp.int32)
    col = edges[1].astype(jnp.int32)
    col4 = jnp.stack([jnp.stack([col + (2 * pair + c) * _N
                                 for pair in range(2)])
                      for c in range(2)])
    col4 = col4.reshape(2, 2, 16, _NB, _B)
    row3 = row.reshape(16, _NB, _B)
    zrows = jnp.zeros((_RPT, 128), jnp.float32)

    sup = _mm(features, W1)
    agg = _spmm(sup, col4, row3, zrows)
    sup = _mm_bias_relu(agg, b1, W2)
    agg = _spmm(sup, col4, row3, zrows)
    sup = _mm_bias_relu(agg, b2, W3)
    agg = _spmm(sup, col4, row3, zrows)
    return _logsoftmax_bias(agg, b3)


# speedup vs baseline: 2.5103x; 1.2532x over previous
import jax
import jax.numpy as jnp
from jax import lax
from jax.experimental import pallas as pl
from jax.experimental.pallas import tpu as pltpu
from jax.experimental.pallas import tpu_sc as plsc

_N = 10000
_E = 160000
_B = 125
_PB = 40
_NPH = 2
_NP = 10112
_RPT = 632
_BM = 1000


def _mm_body(x_ref, w_ref, o_ref):
    o_ref[...] = jnp.dot(x_ref[...], w_ref[...],
                         preferred_element_type=jnp.float32)


def _mm(x, w):
    m, k = x.shape
    n = w.shape[1]
    return pl.pallas_call(
        _mm_body,
        grid=(m // _BM,),
        in_specs=[pl.BlockSpec((_BM, k), lambda i: (i, 0)),
                  pl.BlockSpec((k, n), lambda i: (0, 0))],
        out_specs=pl.BlockSpec((_BM, n), lambda i: (i, 0)),
        out_shape=jax.ShapeDtypeStruct((m, n), jnp.float32),
    )(x, w)


def _mm_bias_relu_body(x_ref, b_ref, w_ref, o_ref):
    h = jnp.maximum(x_ref[...] + b_ref[...], 0.0)
    o_ref[...] = jnp.dot(h, w_ref[...], preferred_element_type=jnp.float32)


def _mm_bias_relu(x, b, w):
    m, k = x.shape
    n = w.shape[1]
    return pl.pallas_call(
        _mm_bias_relu_body,
        grid=(m // _BM,),
        in_specs=[pl.BlockSpec((_BM, k), lambda i: (i, 0)),
                  pl.BlockSpec((1, k), lambda i: (0, 0)),
                  pl.BlockSpec((k, n), lambda i: (0, 0))],
        out_specs=pl.BlockSpec((_BM, n), lambda i: (i, 0)),
        out_shape=jax.ShapeDtypeStruct((m, n), jnp.float32),
    )(x, b.reshape(1, k), w)


def _logsoftmax_bias_body(x_ref, b_ref, o_ref):
    h = x_ref[...] + b_ref[...]
    m = jnp.max(h, axis=1, keepdims=True)
    e = jnp.exp(h - m)
    s = jnp.sum(e, axis=1, keepdims=True)
    o_ref[...] = h - m - jnp.log(s)


def _logsoftmax_bias(x, b):
    m, n = x.shape
    return pl.pallas_call(
        _logsoftmax_bias_body,
        grid=(m // _BM,),
        in_specs=[pl.BlockSpec((_BM, n), lambda i: (i, 0)),
                  pl.BlockSpec((1, n), lambda i: (0, 0))],
        out_specs=pl.BlockSpec((_BM, n), lambda i: (i, 0)),
        out_shape=jax.ShapeDtypeStruct((m, n), jnp.float32),
    )(x, b.reshape(1, n))


_sc_mesh = plsc.VectorSubcoreMesh(core_axis_name="c", subcore_axis_name="s",
                                  num_cores=2)

_sc_scratch = [
    pltpu.VMEM_SHARED((_NP, 128), jnp.float32),
    pltpu.VMEM((_PB, _B), jnp.int32),
    pltpu.VMEM((_PB, _B), jnp.int32),
    pltpu.VMEM((_B, 128), jnp.float32),
    pltpu.VMEM((_B, 128), jnp.float32),
    pltpu.SemaphoreType.DMA,
    pltpu.SemaphoreType.DMA,
]


def _make_sc_spmm(npair):
    def body(sup4, col4, row3, zrows, out4,
             acc, colbuf, rowbuf, b0, b1, semg, sems):
        cid = lax.axis_index("c")
        sid = lax.axis_index("s")
        base = sid * _RPT

        def gather(j, buf):
            return pltpu.async_copy(sup4.at[colbuf.at[j]], buf, semg)

        def gwait(j, buf):
            pltpu.make_async_copy(sup4.at[colbuf.at[j]], buf, semg).wait()

        def scat(j, buf):
            return pltpu.async_copy(buf, acc.at[rowbuf.at[j]], sems,
                                    add=True)

        def swait(j, buf):
            pltpu.make_async_copy(buf, acc.at[rowbuf.at[j]], sems).wait()

        for pair in range(npair):
            pltpu.sync_copy(zrows, acc.at[pl.ds(base, _RPT)])
            plsc.subcore_barrier()
            for p in range(_NPH):
                pltpu.sync_copy(col4.at[cid, pair, sid, p], colbuf)
                pltpu.sync_copy(row3.at[sid, p], rowbuf)
                gather(0, b0)
                gwait(0, b0)
                scat(0, b0)
                gather(1, b1)
                gwait(1, b1)
                scat(1, b1)

                def lbody(j2, carry):
                    j = 2 * j2
                    swait(j - 2, b0)
                    gather(j, b0)
                    gwait(j, b0)
                    scat(j, b0)
                    swait(j - 1, b1)
                    gather(j + 1, b1)
                    gwait(j + 1, b1)
                    scat(j + 1, b1)
                    return carry

                lax.fori_loop(1, _PB // 2, lbody, 0)
                swait(_PB - 2, b0)
                swait(_PB - 1, b1)
            plsc.subcore_barrier()
            pltpu.sync_copy(acc.at[pl.ds(base, _RPT)],
                            out4.at[2 * pair + cid, pl.ds(base, _RPT)])
            if pair + 1 < npair:
                plsc.subcore_barrier()

    return pl.kernel(
        body,
        out_type=jax.ShapeDtypeStruct((2 * npair, _NP, 128), jnp.float32),
        mesh=_sc_mesh,
        scratch_types=_sc_scratch,
    )


_sc_spmm_2 = _make_sc_spmm(2)
_sc_spmm_1 = _make_sc_spmm(1)


def _spmm(sup, col4, row3, zrows):
    d = sup.shape[1]
    nchunk = d // 128
    sup4 = jnp.concatenate(
        [sup[:, k * 128:(k + 1) * 128] for k in range(nchunk)], axis=0)
    fn = _sc_spmm_2 if nchunk == 4 else _sc_spmm_1
    out4 = fn(sup4, col4[:, :nchunk // 2], row3, zrows)
    return jnp.concatenate([out4[k, :_N] for k in range(nchunk)], axis=1)


def kernel(edges, features, W1, b1, W2, b2, W3, b3):
    row = edges[0].astype(jnp.int32)
    col = edges[1].astype(jnp.int32)
    col4 = jnp.stack([jnp.stack([col + (2 * pair + c) * _N
                                 for pair in range(2)])
                      for c in range(2)])
    col4 = col4.reshape(2, 2, 16, _NPH, _PB, _B)
    row3 = row.reshape(16, _NPH, _PB, _B)
    zrows = jnp.zeros((_RPT, 128), jnp.float32)

    sup = _mm(features, W1)
    agg = _spmm(sup, col4, row3, zrows)
    sup = _mm_bias_relu(agg, b1, W2)
    agg = _spmm(sup, col4, row3, zrows)
    sup = _mm_bias_relu(agg, b2, W3)
    agg = _spmm(sup, col4, row3, zrows)
    return _logsoftmax_bias(agg, b3)
